# baseline (speedup 1.0000x reference)
import functools

import numpy as np
import jax
import jax.numpy as jnp
from jax import lax
from jax.experimental import pallas as pl
from jax.experimental.pallas import tpu as pltpu
from jax.experimental.pallas import tpu_sc as plsc

HIDDEN = 64
HPAD = 128
SEQ_LEN = 200
BATCH = 4096

NC = 2
NS = 16
NW = NC * NS
SPW = BATCH // NW
RPW = SPW * SEQ_LEN
IDX_ROWS = RPW // 128
GROUP = 16
NGROUP = SPW // GROUP
IDXBUF_ROWS = 32


def _pieces(s):
    out = []
    pos = 200 * s
    end_seq = pos + 200
    while pos < end_seq:
        k = pos // 128
        end = min((k + 1) * 128, end_seq)
        out.append((k, pos - 128 * k, pos - 200 * s, end - pos))
        pos = end
    return out


def _pe_table() -> np.ndarray:
    seq_pos = np.arange(SEQ_LEN, dtype=np.float32) + 1.0
    power = np.arange(0, HIDDEN, 2, dtype=np.float32) / HIDDEN
    divisor = 10000.0 ** power
    ang = seq_pos[:, None] / divisor[None, :]
    pe = np.stack((np.sin(ang), np.cos(ang)), axis=-1)
    pe = pe.reshape(SEQ_LEN, HIDDEN)
    return np.ascontiguousarray(pe.reshape(SEQ_LEN // 2, 2 * HIDDEN))


_PE = _pe_table()


def _sc_body(idx_hbm, table_hbm, pe_hbm, out_hbm,
             idx_v, rows_v, out_v, pe_v,
             gsem0, gsem1, osem0, osem1, isem0, isem1):
    wid = lax.axis_index("s") * NC + lax.axis_index("c")
    gsem = (gsem0, gsem1)
    osem = (osem0, osem1)
    isem = (isem0, isem1)
    pltpu.sync_copy(pe_hbm, pe_v)

    def idx_desc(o, gb):
        return pltpu.make_async_copy(
            idx_hbm.at[pl.ds(wid * IDX_ROWS + 24 * o, IDXBUF_ROWS)],
            idx_v.at[gb], isem[gb])

    def gather_descs(o, s, b, gb):
        return [
            pltpu.make_async_copy(
                table_hbm.at[idx_v.at[gb, o + k, pl.ds(col, ln)]],
                rows_v.at[b, pl.ds(dst, ln)],
                gsem[b],
            )
            for k, col, dst, ln in _pieces(s)
        ]

    def out_desc(c, b):
        return pltpu.make_async_copy(
            out_v.at[b], out_hbm.at[wid * SPW + c], osem[b])

    pltpu.sync_copy(
        idx_hbm.at[pl.ds(wid * IDX_ROWS, IDXBUF_ROWS)], idx_v.at[0])
    for d in gather_descs(0, 0, 0, 0):
        d.start()

    @pl.loop(0, NGROUP, step=2)
    def _group2(o0):
        for go in range(2):
            o = o0 + go
            gb = go

            @pl.when(o + 1 < NGROUP)
            def _():
                idx_desc(o + 1, 1 - gb).start()

            for s in range(GROUP):
                c = o * GROUP + s
                b = s % 2

                if go == 0 and s == 0:
                    @pl.when(c >= 1)
                    def _():
                        out_desc(c - 1, 1 - b).wait()
                else:
                    out_desc(c - 1, 1 - b).wait()

                if s < GROUP - 1:
                    for d in gather_descs(o, s + 1, 1 - b, gb):
                        d.start()
                else:
                    @pl.when(o + 1 < NGROUP)
                    def _():
                        idx_desc(o + 1, 1 - gb).wait()
                        for d in gather_descs(o + 1, 0, 1 - b, 1 - gb):
                            d.start()

                for d in gather_descs(o, s, b, gb):
                    d.wait()

                @pl.loop(0, SEQ_LEN, unroll=4)
                def _row(r):
                    pr = r // 2
                    pc = (r % 2) * HIDDEN
                    for h in range(HIDDEN // 16):
                        out_v[b, r, pl.ds(h * 16, 16)] = (
                            rows_v[b, r, pl.ds(h * 16, 16)]
                            + pe_v[pr, pl.ds(pc + h * 16, 16)])

                out_desc(c, b).start()

    out_desc(SPW - 1, (SPW - 1) % 2).wait()


@jax.jit
def _sc_embed(idx, table, pe):
    mesh = plsc.VectorSubcoreMesh(
        core_axis_name="c", subcore_axis_name="s", num_cores=NC, num_subcores=NS)
    fn = functools.partial(
        pl.kernel,
        out_type=jax.ShapeDtypeStruct((BATCH, SEQ_LEN, HIDDEN), jnp.float32),
        mesh=mesh,
        scratch_types=[
            pltpu.VMEM((2, IDXBUF_ROWS, 128), jnp.int32),
            pltpu.VMEM((2, SEQ_LEN, HPAD), jnp.float32),
            pltpu.VMEM((2, SEQ_LEN, HIDDEN), jnp.float32),
            pltpu.VMEM((SEQ_LEN // 2, 2 * HIDDEN), jnp.float32),
            pltpu.SemaphoreType.DMA,
            pltpu.SemaphoreType.DMA,
            pltpu.SemaphoreType.DMA,
            pltpu.SemaphoreType.DMA,
            pltpu.SemaphoreType.DMA,
            pltpu.SemaphoreType.DMA,
        ],
        compiler_params=pltpu.CompilerParams(use_tc_tiling_on_sc=True),
    )(_sc_body)
    return fn(idx, table, pe)


def kernel(inputs, table):
    idx = inputs.reshape(BATCH * SEQ_LEN // 128, 128).astype(jnp.int32)
    table = jnp.pad(table.astype(jnp.float32), ((0, 0), (0, HPAD - HIDDEN)))
    pe = jnp.asarray(_PE, dtype=jnp.float32)
    return _sc_embed(idx, table, pe)
